# baseline (speedup 1.0000x reference)
import functools

import jax
import jax.numpy as jnp
from jax import lax
from jax.experimental import pallas as pl
from jax.experimental.pallas import tpu as pltpu
from jax.experimental.pallas import tpu_sc as plsc

VOCAB = 1000000
DIM = 64
LANES = 16

_INFO = plsc.get_sparse_core_info()
_NW = _INFO.num_cores * _INFO.num_subcores

_MESH = plsc.VectorSubcoreMesh(core_axis_name="c", subcore_axis_name="s")
_TC_TILED = pltpu.CompilerParams(
    use_tc_tiling_on_sc=True, needs_layout_passes=False)


VB = 256
NFULL = VOCAB // VB
TAILV = VOCAB - NFULL * VB


def _transpose_body(wt_hbm, tail_hbm, tp_hbm, s_v, tt_v, sem_in, sem_out):
    wid = lax.axis_index("s") * _INFO.num_cores + lax.axis_index("c")

    iota = lax.iota(jnp.int32, LANES)

    def transpose_block(sbuf, tbuf, ncols):
        rowv = []
        colb = []
        for j in range(ncols // LANES):
            c = 16 * j + iota
            rowv.append(c >> 1)
            colb.append((c & 1) * 64)

        def per_f(f, carry):
            for j in range(ncols // LANES):
                vals = sbuf[f, pl.ds(16 * j, LANES)]
                plsc.store_scatter(tbuf, [rowv[j], colb[j] + f], vals)
            return carry

        lax.fori_loop(0, DIM, per_f, 0)

    def fire_read(k, buf):
        pltpu.async_copy(
            wt_hbm.at[:, pl.ds(k * VB, VB)], s_v[buf], sem_in[buf])

    def wait_read(buf):
        pltpu.make_async_copy(
            wt_hbm.at[:, pl.ds(0, VB)], s_v[buf], sem_in[buf]).wait()

    def fire_write(k, buf):
        pltpu.async_copy(
            tt_v[buf], tp_hbm.at[pl.ds(k * (VB // 2), VB // 2)], sem_out[buf])

    def wait_write(buf):
        pltpu.make_async_copy(
            tt_v[buf], tp_hbm.at[pl.ds(0, VB // 2)], sem_out[buf]).wait()

    nk = NFULL // _NW
    rem = NFULL - nk * _NW

    fire_read(wid, 0)

    def step(i2, carry):
        for b in range(2):
            i = 2 * i2 + b
            k = wid + i * _NW

            @pl.when(i + 1 < nk)
            def _():
                fire_read(wid + (i + 1) * _NW, 1 - b)

            wait_read(b)

            @pl.when(i >= 2)
            def _():
                wait_write(b)

            transpose_block(s_v[b], tt_v[b], VB)
            fire_write(k, b)
        return carry

    lax.fori_loop(0, nk // 2, step, 0)
    wait_write(0)
    wait_write(1)

    @pl.when(wid < rem)
    def _():
        k = NFULL - rem + wid
        pltpu.sync_copy(wt_hbm.at[:, pl.ds(k * VB, VB)], s_v[0])
        transpose_block(s_v[0], tt_v[0], VB)
        pltpu.sync_copy(tt_v[0], tp_hbm.at[pl.ds(k * (VB // 2), VB // 2)])

    @pl.when(wid == rem)
    def _():
        pltpu.sync_copy(tail_hbm, tt_v[0].at[pl.ds(0, TAILV // 2)])
        pltpu.sync_copy(tt_v[0].at[pl.ds(0, TAILV // 2)],
                        tp_hbm.at[pl.ds(NFULL * VB // 2, TAILV // 2)])


def _make_transpose():
    return pl.kernel(
        _transpose_body,
        out_type=jax.ShapeDtypeStruct((VOCAB // 2, 128), jnp.float32),
        mesh=_MESH,
        scratch_types=[
            [pltpu.VMEM((DIM, VB), jnp.float32) for _ in range(2)],
            [pltpu.VMEM((VB // 2, 128), jnp.float32) for _ in range(2)],
            [pltpu.SemaphoreType.DMA for _ in range(2)],
            [pltpu.SemaphoreType.DMA for _ in range(2)],
        ],
        compiler_params=_TC_TILED,
    )


SEQ = 200
BATCH = 4096
NGRP = SEQ // 8


def _gather_body(xt_hbm, tp_hbm, op_hbm, idx_v, idxs_v, g_v, o_v,
                 sem_g, sem_o):
    wid = lax.axis_index("s") * _INFO.num_cores + lax.axis_index("c")
    b0 = pl.multiple_of(wid * 128, 128)

    iota = lax.iota(jnp.int32, LANES)
    rowjs = [iota + 16 * j for j in range(8)]

    def load_idx_tile(u):
        pltpu.sync_copy(
            xt_hbm.at[pl.ds(pl.multiple_of(8 * u, 8), 8),
                      pl.ds(b0, 128)], idx_v)
        for r8 in range(8):
            for j in range(8):
                v = idx_v[r8, pl.ds(16 * j, LANES)]
                idxs_v[r8, pl.ds(16 * j, LANES)] = lax.shift_right_logical(
                    v, 1)

    def fire_gather(r, buf):
        pltpu.async_copy(tp_hbm.at[idxs_v.at[r]], g_v[buf], sem_g[buf])

    def wait_gather(buf):
        pltpu.make_async_copy(
            tp_hbm.at[pl.ds(0, 128)], g_v[buf], sem_g[buf]).wait()

    def fire_out(s, buf):
        pltpu.async_copy(
            o_v[buf], op_hbm.at[s, :, pl.ds(b0, 128)], sem_o[buf])

    def wait_out(buf):
        pltpu.make_async_copy(
            o_v[buf], op_hbm.at[0, :, pl.ds(b0, 128)], sem_o[buf]).wait()

    def transpose_select(r, gbuf, obuf):
        for j in range(8):
            hv = (idx_v[r, pl.ds(16 * j, LANES)] & 1) * 64

            def per_f(f, carry):
                vals = plsc.load_gather(g_v[gbuf], [rowjs[j], hv + f])
                o_v[obuf][f, pl.ds(16 * j, LANES)] = vals
                return carry

            lax.fori_loop(0, DIM, per_f, 0)

    def group(u, carry):
        load_idx_tile(u)
        fire_gather(0, 0)
        for r in range(8):
            s = 8 * u + r
            if r + 1 < 8:
                fire_gather(r + 1, (r + 1) % 2)
            wait_gather(r % 2)

            @pl.when((u > 0) | (r >= 2))
            def _():
                wait_out(r % 2)

            transpose_select(r, r % 2, r % 2)
            fire_out(s, r % 2)
        return carry

    lax.fori_loop(0, NGRP, group, 0)
    wait_out(0)
    wait_out(1)


def _make_gather():
    return pl.kernel(
        _gather_body,
        out_type=jax.ShapeDtypeStruct((SEQ, DIM, BATCH), jnp.float32),
        mesh=_MESH,
        scratch_types=[
            pltpu.VMEM((8, 128), jnp.int32),
            pltpu.VMEM((8, 128), jnp.int32),
            [pltpu.VMEM((128, 128), jnp.float32) for _ in range(2)],
            [pltpu.VMEM((DIM, 128), jnp.float32) for _ in range(2)],
            [pltpu.SemaphoreType.DMA for _ in range(2)],
            [pltpu.SemaphoreType.DMA for _ in range(2)],
        ],
        compiler_params=_TC_TILED,
    )


def kernel(x, weight):
    wt = weight.T
    tail = weight[NFULL * VB:].reshape(TAILV // 2, 128)
    tp = _make_transpose()(wt, tail)
    xt = x.T.astype(jnp.int32)
    op = _make_gather()(xt, tp)
    return op.transpose(2, 0, 1)


# speedup vs baseline: 1.5909x; 1.5909x over previous
import functools

import jax
import jax.numpy as jnp
from jax import lax
from jax.experimental import pallas as pl
from jax.experimental.pallas import tpu as pltpu
from jax.experimental.pallas import tpu_sc as plsc

VOCAB = 1000000
DIM = 64
LANES = 16

_INFO = plsc.get_sparse_core_info()
_NW = _INFO.num_cores * _INFO.num_subcores

_MESH = plsc.VectorSubcoreMesh(core_axis_name="c", subcore_axis_name="s")
_TC_TILED = pltpu.CompilerParams(
    use_tc_tiling_on_sc=True, needs_layout_passes=False)


VB = 256
NFULL = VOCAB // VB
TAILV = VOCAB - NFULL * VB


def _transpose_body(wt_hbm, tail_hbm, tp_hbm, s_v, tt_v, sem_in, sem_out):
    wid = lax.axis_index("s") * _INFO.num_cores + lax.axis_index("c")

    iota = lax.iota(jnp.int32, LANES)

    def transpose_block(sbuf, tbuf, ncols):
        rowv = []
        colb = []
        for j in range(ncols // LANES):
            c = 16 * j + iota
            rowv.append(c >> 1)
            colb.append((c & 1) * 64)

        @plsc.parallel_loop(0, DIM, unroll=4)
        def _(f):
            for j in range(ncols // LANES):
                vals = sbuf[f, pl.ds(16 * j, LANES)]
                plsc.store_scatter(tbuf, [rowv[j], colb[j] + f], vals)

    def fire_read(k, buf):
        pltpu.async_copy(
            wt_hbm.at[:, pl.ds(k * VB, VB)], s_v[buf], sem_in[buf])

    def wait_read(buf):
        pltpu.make_async_copy(
            wt_hbm.at[:, pl.ds(0, VB)], s_v[buf], sem_in[buf]).wait()

    def fire_write(k, buf):
        pltpu.async_copy(
            tt_v[buf], tp_hbm.at[pl.ds(k * (VB // 2), VB // 2)], sem_out[buf])

    def wait_write(buf):
        pltpu.make_async_copy(
            tt_v[buf], tp_hbm.at[pl.ds(0, VB // 2)], sem_out[buf]).wait()

    nk = NFULL // _NW
    rem = NFULL - nk * _NW

    fire_read(wid, 0)

    def step(i2, carry):
        for b in range(2):
            i = 2 * i2 + b
            k = wid + i * _NW

            @pl.when(i + 1 < nk)
            def _():
                fire_read(wid + (i + 1) * _NW, 1 - b)

            wait_read(b)

            @pl.when(i >= 2)
            def _():
                wait_write(b)

            transpose_block(s_v[b], tt_v[b], VB)
            fire_write(k, b)
        return carry

    lax.fori_loop(0, nk // 2, step, 0)
    wait_write(0)
    wait_write(1)

    @pl.when(wid < rem)
    def _():
        k = NFULL - rem + wid
        pltpu.sync_copy(wt_hbm.at[:, pl.ds(k * VB, VB)], s_v[0])
        transpose_block(s_v[0], tt_v[0], VB)
        pltpu.sync_copy(tt_v[0], tp_hbm.at[pl.ds(k * (VB // 2), VB // 2)])

    @pl.when(wid == rem)
    def _():
        pltpu.sync_copy(tail_hbm, tt_v[0].at[pl.ds(0, TAILV // 2)])
        pltpu.sync_copy(tt_v[0].at[pl.ds(0, TAILV // 2)],
                        tp_hbm.at[pl.ds(NFULL * VB // 2, TAILV // 2)])


def _make_transpose():
    return pl.kernel(
        _transpose_body,
        out_type=jax.ShapeDtypeStruct((VOCAB // 2, 128), jnp.float32),
        mesh=_MESH,
        scratch_types=[
            [pltpu.VMEM((DIM, VB), jnp.float32) for _ in range(2)],
            [pltpu.VMEM((VB // 2, 128), jnp.float32) for _ in range(2)],
            [pltpu.SemaphoreType.DMA for _ in range(2)],
            [pltpu.SemaphoreType.DMA for _ in range(2)],
        ],
        compiler_params=_TC_TILED,
    )


SEQ = 200
BATCH = 4096
NGRP = SEQ // 8


def _gather_body(xt_hbm, tp_hbm, op_hbm, idx_v, idxs_v, g_v, o_v,
                 sem_g, sem_o):
    wid = lax.axis_index("s") * _INFO.num_cores + lax.axis_index("c")
    b0 = pl.multiple_of(wid * 128, 128)

    iota = lax.iota(jnp.int32, LANES)
    rowjs = [iota + 16 * j for j in range(8)]

    def load_idx_tile(u):
        pltpu.sync_copy(
            xt_hbm.at[pl.ds(pl.multiple_of(8 * u, 8), 8),
                      pl.ds(b0, 128)], idx_v)
        for r8 in range(8):
            for j in range(8):
                v = idx_v[r8, pl.ds(16 * j, LANES)]
                idxs_v[r8, pl.ds(16 * j, LANES)] = lax.shift_right_logical(
                    v, 1)

    def fire_gather(r, buf):
        pltpu.async_copy(tp_hbm.at[idxs_v.at[r]], g_v[buf], sem_g[buf])

    def wait_gather(buf):
        pltpu.make_async_copy(
            tp_hbm.at[pl.ds(0, 128)], g_v[buf], sem_g[buf]).wait()

    def fire_out(s, buf):
        pltpu.async_copy(
            o_v[buf], op_hbm.at[s, :, pl.ds(b0, 128)], sem_o[buf])

    def wait_out(buf):
        pltpu.make_async_copy(
            o_v[buf], op_hbm.at[0, :, pl.ds(b0, 128)], sem_o[buf]).wait()

    def transpose_select(r, gbuf, obuf):
        hvs = [(idx_v[r, pl.ds(16 * j, LANES)] & 1) * 64 for j in range(8)]

        @plsc.parallel_loop(0, DIM, unroll=4)
        def _(f):
            for j in range(8):
                vals = plsc.load_gather(g_v[gbuf], [rowjs[j], hvs[j] + f])
                o_v[obuf][f, pl.ds(16 * j, LANES)] = vals

    def group(u, carry):
        load_idx_tile(u)
        fire_gather(0, 0)
        for r in range(8):
            s = 8 * u + r
            if r + 1 < 8:
                fire_gather(r + 1, (r + 1) % 2)
            wait_gather(r % 2)

            @pl.when((u > 0) | (r >= 2))
            def _():
                wait_out(r % 2)

            transpose_select(r, r % 2, r % 2)
            fire_out(s, r % 2)
        return carry

    lax.fori_loop(0, NGRP, group, 0)
    wait_out(0)
    wait_out(1)


def _make_gather():
    return pl.kernel(
        _gather_body,
        out_type=jax.ShapeDtypeStruct((SEQ, DIM, BATCH), jnp.float32),
        mesh=_MESH,
        scratch_types=[
            pltpu.VMEM((8, 128), jnp.int32),
            pltpu.VMEM((8, 128), jnp.int32),
            [pltpu.VMEM((128, 128), jnp.float32) for _ in range(2)],
            [pltpu.VMEM((DIM, 128), jnp.float32) for _ in range(2)],
            [pltpu.SemaphoreType.DMA for _ in range(2)],
            [pltpu.SemaphoreType.DMA for _ in range(2)],
        ],
        compiler_params=_TC_TILED,
    )


def kernel(x, weight):
    wt = weight.T
    tail = weight[NFULL * VB:].reshape(TAILV // 2, 128)
    tp = _make_transpose()(wt, tail)
    xt = x.T.astype(jnp.int32)
    op = _make_gather()(xt, tp)
    return op.transpose(2, 0, 1)


# speedup vs baseline: 3.9128x; 2.4595x over previous
import functools

import jax
import jax.numpy as jnp
from jax import lax
from jax.experimental import pallas as pl
from jax.experimental.pallas import tpu as pltpu
from jax.experimental.pallas import tpu_sc as plsc

VOCAB = 1000000
DIM = 64
LANES = 16

_INFO = plsc.get_sparse_core_info()
_NW = _INFO.num_cores * _INFO.num_subcores

_MESH = plsc.VectorSubcoreMesh(core_axis_name="c", subcore_axis_name="s")
_TC_TILED = pltpu.CompilerParams(
    use_tc_tiling_on_sc=True, needs_layout_passes=False)


VB = 256
NFULL = VOCAB // VB
TAILV = VOCAB - NFULL * VB


def _transpose_body(wt_hbm, tail_hbm, tp_hbm, s_v, tt_v, sem_in, sem_out):
    wid = lax.axis_index("s") * _INFO.num_cores + lax.axis_index("c")

    iota = lax.iota(jnp.int32, LANES)

    def transpose_block(sbuf, tbuf, ncols):
        rowv = [(16 * j + iota) >> 1 for j in range(ncols // LANES)]
        colv = [16 * j + iota for j in range(ncols // LANES)]
        hv = (iota & 1) * 64

        @plsc.parallel_loop(0, LANES, unroll=2)
        def _(d):
            fd = (iota + d) & (LANES - 1)
            for k in range(DIM // LANES):
                fv = fd + 16 * k
                for j in range(ncols // LANES):
                    vals = plsc.load_gather(sbuf, [fv, colv[j]])
                    plsc.store_scatter(tbuf, [rowv[j], hv + fv], vals)

    def fire_read(k, buf):
        pltpu.async_copy(
            wt_hbm.at[:, pl.ds(k * VB, VB)], s_v[buf], sem_in[buf])

    def wait_read(buf):
        pltpu.make_async_copy(
            wt_hbm.at[:, pl.ds(0, VB)], s_v[buf], sem_in[buf]).wait()

    def fire_write(k, buf):
        pltpu.async_copy(
            tt_v[buf], tp_hbm.at[pl.ds(k * (VB // 2), VB // 2)], sem_out[buf])

    def wait_write(buf):
        pltpu.make_async_copy(
            tt_v[buf], tp_hbm.at[pl.ds(0, VB // 2)], sem_out[buf]).wait()

    nk = NFULL // _NW
    rem = NFULL - nk * _NW

    fire_read(wid, 0)

    def step(i2, carry):
        for b in range(2):
            i = 2 * i2 + b
            k = wid + i * _NW

            @pl.when(i + 1 < nk)
            def _():
                fire_read(wid + (i + 1) * _NW, 1 - b)

            wait_read(b)

            @pl.when(i >= 2)
            def _():
                wait_write(b)

            transpose_block(s_v[b], tt_v[b], VB)
            fire_write(k, b)
        return carry

    lax.fori_loop(0, nk // 2, step, 0)
    wait_write(0)
    wait_write(1)

    @pl.when(wid < rem)
    def _():
        k = NFULL - rem + wid
        pltpu.sync_copy(wt_hbm.at[:, pl.ds(k * VB, VB)], s_v[0])
        transpose_block(s_v[0], tt_v[0], VB)
        pltpu.sync_copy(tt_v[0], tp_hbm.at[pl.ds(k * (VB // 2), VB // 2)])

    @pl.when(wid == rem)
    def _():
        pltpu.sync_copy(tail_hbm, tt_v[0].at[pl.ds(0, TAILV // 2)])
        pltpu.sync_copy(tt_v[0].at[pl.ds(0, TAILV // 2)],
                        tp_hbm.at[pl.ds(NFULL * VB // 2, TAILV // 2)])


def _make_transpose():
    return pl.kernel(
        _transpose_body,
        out_type=jax.ShapeDtypeStruct((VOCAB // 2, 128), jnp.float32),
        mesh=_MESH,
        scratch_types=[
            [pltpu.VMEM((DIM, VB), jnp.float32) for _ in range(2)],
            [pltpu.VMEM((VB // 2, 128), jnp.float32) for _ in range(2)],
            [pltpu.SemaphoreType.DMA for _ in range(2)],
            [pltpu.SemaphoreType.DMA for _ in range(2)],
        ],
        compiler_params=_TC_TILED,
    )


SEQ = 200
BATCH = 4096
NGRP = SEQ // 8


def _gather_body(xt_hbm, tp_hbm, op_hbm, idx_v, idxs_v, g_v, o_v,
                 sem_g, sem_o):
    wid = lax.axis_index("s") * _INFO.num_cores + lax.axis_index("c")
    b0 = pl.multiple_of(wid * 128, 128)

    iota = lax.iota(jnp.int32, LANES)
    rowjs = [iota + 16 * j for j in range(8)]

    def load_idx_tile(u):
        pltpu.sync_copy(
            xt_hbm.at[pl.ds(pl.multiple_of(8 * u, 8), 8),
                      pl.ds(b0, 128)], idx_v)
        for r8 in range(8):
            for j in range(8):
                v = idx_v[r8, pl.ds(16 * j, LANES)]
                idxs_v[r8, pl.ds(16 * j, LANES)] = lax.shift_right_logical(
                    v, 1)

    def fire_gather(r, buf):
        pltpu.async_copy(tp_hbm.at[idxs_v.at[r]], g_v[buf], sem_g[buf])

    def wait_gather(buf):
        pltpu.make_async_copy(
            tp_hbm.at[pl.ds(0, 128)], g_v[buf], sem_g[buf]).wait()

    def fire_out(s, buf):
        pltpu.async_copy(
            o_v[buf], op_hbm.at[s, :, pl.ds(b0, 128)], sem_o[buf])

    def wait_out(buf):
        pltpu.make_async_copy(
            o_v[buf], op_hbm.at[0, :, pl.ds(b0, 128)], sem_o[buf]).wait()

    def transpose_select(r, gbuf, obuf):
        hvs = [(idx_v[r, pl.ds(16 * j, LANES)] & 1) * 64 for j in range(8)]

        @plsc.parallel_loop(0, LANES, unroll=2)
        def _(d):
            fd = (iota + d) & (LANES - 1)
            for k in range(DIM // LANES):
                fv = fd + 16 * k
                for j in range(8):
                    vals = plsc.load_gather(g_v[gbuf], [rowjs[j], hvs[j] + fv])
                    plsc.store_scatter(o_v[obuf], [fv, rowjs[j]], vals)

    def group(u, carry):
        load_idx_tile(u)
        fire_gather(0, 0)
        for r in range(8):
            s = 8 * u + r
            if r + 1 < 8:
                fire_gather(r + 1, (r + 1) % 2)
            wait_gather(r % 2)

            @pl.when((u > 0) | (r >= 2))
            def _():
                wait_out(r % 2)

            transpose_select(r, r % 2, r % 2)
            fire_out(s, r % 2)
        return carry

    lax.fori_loop(0, NGRP, group, 0)
    wait_out(0)
    wait_out(1)


def _make_gather():
    return pl.kernel(
        _gather_body,
        out_type=jax.ShapeDtypeStruct((SEQ, DIM, BATCH), jnp.float32),
        mesh=_MESH,
        scratch_types=[
            pltpu.VMEM((8, 128), jnp.int32),
            pltpu.VMEM((8, 128), jnp.int32),
            [pltpu.VMEM((128, 128), jnp.float32) for _ in range(2)],
            [pltpu.VMEM((DIM, 128), jnp.float32) for _ in range(2)],
            [pltpu.SemaphoreType.DMA for _ in range(2)],
            [pltpu.SemaphoreType.DMA for _ in range(2)],
        ],
        compiler_params=_TC_TILED,
    )


def kernel(x, weight):
    wt = weight.T
    tail = weight[NFULL * VB:].reshape(TAILV // 2, 128)
    tp = _make_transpose()(wt, tail)
    xt = x.T.astype(jnp.int32)
    op = _make_gather()(xt, tp)
    return op.transpose(2, 0, 1)


# speedup vs baseline: 4.7250x; 1.2076x over previous
import functools

import jax
import jax.numpy as jnp
from jax import lax
from jax.experimental import pallas as pl
from jax.experimental.pallas import tpu as pltpu
from jax.experimental.pallas import tpu_sc as plsc

VOCAB = 1000000
DIM = 64
LANES = 16

_INFO = plsc.get_sparse_core_info()
_NW = _INFO.num_cores * _INFO.num_subcores

_MESH = plsc.VectorSubcoreMesh(core_axis_name="c", subcore_axis_name="s")
_TC_TILED = pltpu.CompilerParams(
    use_tc_tiling_on_sc=True, needs_layout_passes=False)


VB = 256
NFULL = VOCAB // VB
TAILV = VOCAB - NFULL * VB


def _transpose_body(wt_hbm, tail_hbm, tp_hbm, s_v, tt_v, sem_in, sem_out):
    wid = lax.axis_index("s") * _INFO.num_cores + lax.axis_index("c")

    iota = lax.iota(jnp.int32, LANES)

    def transpose_block(sbuf, tbuf, ncols):
        rowv = [(16 * j + iota) >> 1 for j in range(ncols // LANES)]
        colv = [16 * j + iota for j in range(ncols // LANES)]
        hv = (iota & 1) * 64

        @plsc.parallel_loop(0, LANES, unroll=4)
        def _(d):
            fd = (iota + d) & (LANES - 1)
            for k in range(DIM // LANES):
                fv = fd + 16 * k
                for j in range(ncols // LANES):
                    vals = plsc.load_gather(sbuf, [fv, colv[j]])
                    plsc.store_scatter(tbuf, [rowv[j], hv + fv], vals)

    def fire_read(k, buf):
        pltpu.async_copy(
            wt_hbm.at[:, pl.ds(k * VB, VB)], s_v[buf], sem_in[buf])

    def wait_read(buf):
        pltpu.make_async_copy(
            wt_hbm.at[:, pl.ds(0, VB)], s_v[buf], sem_in[buf]).wait()

    def fire_write(k, buf):
        pltpu.async_copy(
            tt_v[buf], tp_hbm.at[pl.ds(k * (VB // 2), VB // 2)], sem_out[buf])

    def wait_write(buf):
        pltpu.make_async_copy(
            tt_v[buf], tp_hbm.at[pl.ds(0, VB // 2)], sem_out[buf]).wait()

    nk = NFULL // _NW
    rem = NFULL - nk * _NW

    fire_read(wid, 0)

    def step(i2, carry):
        for b in range(2):
            i = 2 * i2 + b
            k = wid + i * _NW

            @pl.when(i + 1 < nk)
            def _():
                fire_read(wid + (i + 1) * _NW, 1 - b)

            wait_read(b)

            @pl.when(i >= 2)
            def _():
                wait_write(b)

            transpose_block(s_v[b], tt_v[b], VB)
            fire_write(k, b)
        return carry

    lax.fori_loop(0, nk // 2, step, 0)
    wait_write(0)
    wait_write(1)

    @pl.when(wid < rem)
    def _():
        k = NFULL - rem + wid
        pltpu.sync_copy(wt_hbm.at[:, pl.ds(k * VB, VB)], s_v[0])
        transpose_block(s_v[0], tt_v[0], VB)
        pltpu.sync_copy(tt_v[0], tp_hbm.at[pl.ds(k * (VB // 2), VB // 2)])

    @pl.when(wid == rem)
    def _():
        pltpu.sync_copy(tail_hbm, tt_v[0].at[pl.ds(0, TAILV // 2)])
        pltpu.sync_copy(tt_v[0].at[pl.ds(0, TAILV // 2)],
                        tp_hbm.at[pl.ds(NFULL * VB // 2, TAILV // 2)])


def _make_transpose():
    return pl.kernel(
        _transpose_body,
        out_type=jax.ShapeDtypeStruct((VOCAB // 2, 128), jnp.float32),
        mesh=_MESH,
        scratch_types=[
            [pltpu.VMEM((DIM, VB), jnp.float32) for _ in range(2)],
            [pltpu.VMEM((VB // 2, 128), jnp.float32) for _ in range(2)],
            [pltpu.SemaphoreType.DMA for _ in range(2)],
            [pltpu.SemaphoreType.DMA for _ in range(2)],
        ],
        compiler_params=_TC_TILED,
    )


SEQ = 200
BATCH = 4096
NGRP = SEQ // 8


def _gather_body(xt_hbm, tp_hbm, op_hbm, idx_v, idxs_v, g_v, o_v,
                 sem_g, sem_o):
    wid = lax.axis_index("s") * _INFO.num_cores + lax.axis_index("c")
    b0 = pl.multiple_of(wid * 128, 128)

    iota = lax.iota(jnp.int32, LANES)
    rowjs = [iota + 16 * j for j in range(8)]

    def load_idx_tile(u):
        pltpu.sync_copy(
            xt_hbm.at[pl.ds(pl.multiple_of(8 * u, 8), 8),
                      pl.ds(b0, 128)], idx_v)
        for r8 in range(8):
            for j in range(8):
                v = idx_v[r8, pl.ds(16 * j, LANES)]
                idxs_v[r8, pl.ds(16 * j, LANES)] = lax.shift_right_logical(
                    v, 1)

    def fire_gather(r, buf):
        pltpu.async_copy(tp_hbm.at[idxs_v.at[r]], g_v[buf], sem_g[buf])

    def wait_gather(buf):
        pltpu.make_async_copy(
            tp_hbm.at[pl.ds(0, 128)], g_v[buf], sem_g[buf]).wait()

    def fire_out(s, buf):
        pltpu.async_copy(
            o_v[buf], op_hbm.at[s, :, pl.ds(b0, 128)], sem_o[buf])

    def wait_out(buf):
        pltpu.make_async_copy(
            o_v[buf], op_hbm.at[0, :, pl.ds(b0, 128)], sem_o[buf]).wait()

    def transpose_select(r, gbuf, obuf):
        hvs = [(idx_v[r, pl.ds(16 * j, LANES)] & 1) * 64 for j in range(8)]

        @plsc.parallel_loop(0, LANES, unroll=4)
        def _(d):
            fd = (iota + d) & (LANES - 1)
            for k in range(DIM // LANES):
                fv = fd + 16 * k
                for j in range(8):
                    vals = plsc.load_gather(g_v[gbuf], [rowjs[j], hvs[j] + fv])
                    plsc.store_scatter(o_v[obuf], [fv, rowjs[j]], vals)

    def group(u, carry):
        load_idx_tile(u)
        fire_gather(0, 0)
        for r in range(8):
            s = 8 * u + r
            if r + 1 < 8:
                fire_gather(r + 1, (r + 1) % 2)
            wait_gather(r % 2)

            @pl.when((u > 0) | (r >= 2))
            def _():
                wait_out(r % 2)

            transpose_select(r, r % 2, r % 2)
            fire_out(s, r % 2)
        return carry

    lax.fori_loop(0, NGRP, group, 0)
    wait_out(0)
    wait_out(1)


def _make_gather():
    return pl.kernel(
        _gather_body,
        out_type=jax.ShapeDtypeStruct((SEQ, DIM, BATCH), jnp.float32),
        mesh=_MESH,
        scratch_types=[
            pltpu.VMEM((8, 128), jnp.int32),
            pltpu.VMEM((8, 128), jnp.int32),
            [pltpu.VMEM((128, 128), jnp.float32) for _ in range(2)],
            [pltpu.VMEM((DIM, 128), jnp.float32) for _ in range(2)],
            [pltpu.SemaphoreType.DMA for _ in range(2)],
            [pltpu.SemaphoreType.DMA for _ in range(2)],
        ],
        compiler_params=_TC_TILED,
    )


def kernel(x, weight):
    wt = weight.T
    tail = weight[NFULL * VB:].reshape(TAILV // 2, 128)
    tp = _make_transpose()(wt, tail)
    xt = x.T.astype(jnp.int32)
    op = _make_gather()(xt, tp)
    return op.transpose(2, 0, 1)


# speedup vs baseline: 4.7800x; 1.0117x over previous
import functools

import jax
import jax.numpy as jnp
from jax import lax
from jax.experimental import pallas as pl
from jax.experimental.pallas import tpu as pltpu
from jax.experimental.pallas import tpu_sc as plsc

VOCAB = 1000000
DIM = 64
LANES = 16

_INFO = plsc.get_sparse_core_info()
_NW = _INFO.num_cores * _INFO.num_subcores

_MESH = plsc.VectorSubcoreMesh(core_axis_name="c", subcore_axis_name="s")
_TC_TILED = pltpu.CompilerParams(
    use_tc_tiling_on_sc=True, needs_layout_passes=False)


VB = 256
NFULL = VOCAB // VB
TAILV = VOCAB - NFULL * VB


def _transpose_body(wt_hbm, tail_hbm, tp_hbm, s_v, tt_v, sem_in, sem_out):
    wid = lax.axis_index("s") * _INFO.num_cores + lax.axis_index("c")

    iota = lax.iota(jnp.int32, LANES)

    def transpose_block(sbuf, tbuf, ncols):
        rowv = [(16 * j + iota) >> 1 for j in range(ncols // LANES)]
        colv = [16 * j + iota for j in range(ncols // LANES)]
        hv = (iota & 1) * 64

        @plsc.parallel_loop(0, LANES, unroll=4)
        def _(d):
            fd = (iota + d) & (LANES - 1)
            for k in range(DIM // LANES):
                fv = fd + 16 * k
                for j in range(ncols // LANES):
                    vals = plsc.load_gather(sbuf, [fv, colv[j]])
                    plsc.store_scatter(tbuf, [rowv[j], hv + fv], vals)

    def fire_read(k, buf):
        pltpu.async_copy(
            wt_hbm.at[:, pl.ds(k * VB, VB)], s_v[buf], sem_in[buf])

    def wait_read(buf):
        pltpu.make_async_copy(
            wt_hbm.at[:, pl.ds(0, VB)], s_v[buf], sem_in[buf]).wait()

    def fire_write(k, buf):
        pltpu.async_copy(
            tt_v[buf], tp_hbm.at[pl.ds(k * (VB // 2), VB // 2)], sem_out[buf])

    def wait_write(buf):
        pltpu.make_async_copy(
            tt_v[buf], tp_hbm.at[pl.ds(0, VB // 2)], sem_out[buf]).wait()

    nk = NFULL // _NW
    rem = NFULL - nk * _NW

    fire_read(wid, 0)

    def step(i2, carry):
        for b in range(2):
            i = 2 * i2 + b
            k = wid + i * _NW

            @pl.when(i + 1 < nk)
            def _():
                fire_read(wid + (i + 1) * _NW, 1 - b)

            wait_read(b)

            @pl.when(i >= 2)
            def _():
                wait_write(b)

            transpose_block(s_v[b], tt_v[b], VB)
            fire_write(k, b)
        return carry

    lax.fori_loop(0, nk // 2, step, 0)
    wait_write(0)
    wait_write(1)

    @pl.when(wid < rem)
    def _():
        k = NFULL - rem + wid
        pltpu.sync_copy(wt_hbm.at[:, pl.ds(k * VB, VB)], s_v[0])
        transpose_block(s_v[0], tt_v[0], VB)
        pltpu.sync_copy(tt_v[0], tp_hbm.at[pl.ds(k * (VB // 2), VB // 2)])

    @pl.when(wid == rem)
    def _():
        pltpu.sync_copy(tail_hbm, tt_v[0].at[pl.ds(0, TAILV // 2)])
        pltpu.sync_copy(tt_v[0].at[pl.ds(0, TAILV // 2)],
                        tp_hbm.at[pl.ds(NFULL * VB // 2, TAILV // 2)])


def _make_transpose():
    return pl.kernel(
        _transpose_body,
        out_type=jax.ShapeDtypeStruct((VOCAB // 2, 128), jnp.float32),
        mesh=_MESH,
        scratch_types=[
            [pltpu.VMEM((DIM, VB), jnp.float32) for _ in range(2)],
            [pltpu.VMEM((VB // 2, 128), jnp.float32) for _ in range(2)],
            [pltpu.SemaphoreType.DMA for _ in range(2)],
            [pltpu.SemaphoreType.DMA for _ in range(2)],
        ],
        compiler_params=_TC_TILED,
    )


SEQ = 200
BATCH = 4096
NGRP = SEQ // 8


def _gather_body(xt_hbm, tp_hbm, op_hbm, idx_v, idxs_v, g_v, o_v,
                 sem_g, sem_o):
    wid = lax.axis_index("s") * _INFO.num_cores + lax.axis_index("c")
    b0 = pl.multiple_of(wid * 128, 128)

    iota = lax.iota(jnp.int32, LANES)
    rowjs = [iota + 16 * j for j in range(8)]

    def load_idx_tile(u):
        pltpu.sync_copy(
            xt_hbm.at[pl.ds(pl.multiple_of(8 * u, 8), 8),
                      pl.ds(b0, 128)], idx_v)
        for r8 in range(8):
            for j in range(8):
                v = idx_v[r8, pl.ds(16 * j, LANES)]
                idxs_v[pl.ds(r8 * 128 + 16 * j, LANES)] = (
                    lax.shift_right_logical(v, 1))

    def fire_gather(p, buf):
        pltpu.async_copy(tp_hbm.at[idxs_v.at[pl.ds(256 * p, 256)]],
                         g_v[buf], sem_g[buf])

    def wait_gather(buf):
        pltpu.make_async_copy(
            tp_hbm.at[pl.ds(0, 256)], g_v[buf], sem_g[buf]).wait()

    def fire_out(s, buf):
        pltpu.async_copy(
            o_v[buf], op_hbm.at[s, :, pl.ds(b0, 128)], sem_o[buf])

    def wait_out(buf):
        pltpu.make_async_copy(
            o_v[buf], op_hbm.at[0, :, pl.ds(b0, 128)], sem_o[buf]).wait()

    def transpose_select(r, par, gbuf, obuf):
        hvs = [(idx_v[r, pl.ds(16 * j, LANES)] & 1) * 64 for j in range(8)]
        growjs = [rowjs[j] + 128 * par for j in range(8)]

        @plsc.parallel_loop(0, LANES, unroll=4)
        def _(d):
            fd = (iota + d) & (LANES - 1)
            for k in range(DIM // LANES):
                fv = fd + 16 * k
                for j in range(8):
                    vals = plsc.load_gather(g_v[gbuf], [growjs[j], hvs[j] + fv])
                    plsc.store_scatter(o_v[obuf], [fv, rowjs[j]], vals)

    def group(u, carry):
        load_idx_tile(u)
        fire_gather(0, 0)
        for p in range(4):
            if p + 1 < 4:
                fire_gather(p + 1, (p + 1) % 2)
            wait_gather(p % 2)
            for par in range(2):
                s = 8 * u + 2 * p + par

                @pl.when((u > 0) | (p >= 1))
                def _():
                    wait_out(par)

                transpose_select(2 * p + par, par, p % 2, par)
                fire_out(s, par)
        return carry

    lax.fori_loop(0, NGRP, group, 0)
    wait_out(0)
    wait_out(1)


def _make_gather():
    return pl.kernel(
        _gather_body,
        out_type=jax.ShapeDtypeStruct((SEQ, DIM, BATCH), jnp.float32),
        mesh=_MESH,
        scratch_types=[
            pltpu.VMEM((8, 128), jnp.int32),
            pltpu.VMEM((1024,), jnp.int32),
            [pltpu.VMEM((256, 128), jnp.float32) for _ in range(2)],
            [pltpu.VMEM((DIM, 128), jnp.float32) for _ in range(2)],
            [pltpu.SemaphoreType.DMA for _ in range(2)],
            [pltpu.SemaphoreType.DMA for _ in range(2)],
        ],
        compiler_params=_TC_TILED,
    )


def kernel(x, weight):
    wt = weight.T
    tail = weight[NFULL * VB:].reshape(TAILV // 2, 128)
    tp = _make_transpose()(wt, tail)
    xt = x.T.astype(jnp.int32)
    op = _make_gather()(xt, tp)
    return op.transpose(2, 0, 1)


# speedup vs baseline: 4.8839x; 1.0217x over previous
import functools

import jax
import jax.numpy as jnp
from jax import lax
from jax.experimental import pallas as pl
from jax.experimental.pallas import tpu as pltpu
from jax.experimental.pallas import tpu_sc as plsc

VOCAB = 1000000
DIM = 64
LANES = 16

_INFO = plsc.get_sparse_core_info()
_NW = _INFO.num_cores * _INFO.num_subcores

_MESH = plsc.VectorSubcoreMesh(core_axis_name="c", subcore_axis_name="s")
_TC_TILED = pltpu.CompilerParams(
    use_tc_tiling_on_sc=True, needs_layout_passes=False)


VB = 256
NFULL = VOCAB // VB
TAILV = VOCAB - NFULL * VB


def _transpose_body(wt_hbm, tail_hbm, tp_hbm, s_v, tt_v, sem_in, sem_out):
    wid = lax.axis_index("s") * _INFO.num_cores + lax.axis_index("c")

    iota = lax.iota(jnp.int32, LANES)

    def transpose_block(sbuf, tbuf, ncols):
        rowv = [(16 * j + iota) >> 1 for j in range(ncols // LANES)]
        colv = [16 * j + iota for j in range(ncols // LANES)]
        hv = (iota & 1) * 64

        @plsc.parallel_loop(0, LANES, unroll=4)
        def _(d):
            fd = (iota + d) & (LANES - 1)
            for k in range(DIM // LANES):
                fv = fd + 16 * k
                for j in range(ncols // LANES):
                    vals = plsc.load_gather(sbuf, [fv, colv[j]])
                    plsc.store_scatter(tbuf, [rowv[j], hv + fv], vals)

    def fire_read(k, buf):
        pltpu.async_copy(
            wt_hbm.at[:, pl.ds(k * VB, VB)], s_v[buf], sem_in[buf])

    def wait_read(buf):
        pltpu.make_async_copy(
            wt_hbm.at[:, pl.ds(0, VB)], s_v[buf], sem_in[buf]).wait()

    def fire_write(k, buf):
        pltpu.async_copy(
            tt_v[buf], tp_hbm.at[pl.ds(k * (VB // 2), VB // 2)], sem_out[buf])

    def wait_write(buf):
        pltpu.make_async_copy(
            tt_v[buf], tp_hbm.at[pl.ds(0, VB // 2)], sem_out[buf]).wait()

    nk = NFULL // _NW
    rem = NFULL - nk * _NW

    fire_read(wid, 0)

    def step(i2, carry):
        for b in range(2):
            i = 2 * i2 + b
            k = wid + i * _NW

            @pl.when(i + 1 < nk)
            def _():
                fire_read(wid + (i + 1) * _NW, 1 - b)

            wait_read(b)

            @pl.when(i >= 2)
            def _():
                wait_write(b)

            transpose_block(s_v[b], tt_v[b], VB)
            fire_write(k, b)
        return carry

    lax.fori_loop(0, nk // 2, step, 0)
    wait_write(0)
    wait_write(1)

    @pl.when(wid < rem)
    def _():
        k = NFULL - rem + wid
        pltpu.sync_copy(wt_hbm.at[:, pl.ds(k * VB, VB)], s_v[0])
        transpose_block(s_v[0], tt_v[0], VB)
        pltpu.sync_copy(tt_v[0], tp_hbm.at[pl.ds(k * (VB // 2), VB // 2)])

    @pl.when(wid == rem)
    def _():
        pltpu.sync_copy(tail_hbm, tt_v[0].at[pl.ds(0, TAILV // 2)])
        pltpu.sync_copy(tt_v[0].at[pl.ds(0, TAILV // 2)],
                        tp_hbm.at[pl.ds(NFULL * VB // 2, TAILV // 2)])


def _make_transpose():
    return pl.kernel(
        _transpose_body,
        out_type=jax.ShapeDtypeStruct((VOCAB // 2, 128), jnp.float32),
        mesh=_MESH,
        scratch_types=[
            [pltpu.VMEM((DIM, VB), jnp.float32) for _ in range(2)],
            [pltpu.VMEM((VB // 2, 128), jnp.float32) for _ in range(2)],
            [pltpu.SemaphoreType.DMA for _ in range(2)],
            [pltpu.SemaphoreType.DMA for _ in range(2)],
        ],
        compiler_params=_TC_TILED,
    )


SEQ = 200
BATCH = 4096
NGRP = SEQ // 8


def _gather_body(xt_hbm, tp_hbm, op_hbm, idx_v, idxs_v, g_v, o_v,
                 sem_g, sem_o):
    wid = lax.axis_index("s") * _INFO.num_cores + lax.axis_index("c")
    b0 = pl.multiple_of(wid * 128, 128)

    iota = lax.iota(jnp.int32, LANES)
    rowjs = [iota + 16 * j for j in range(8)]

    def load_idx_tile(u):
        pltpu.sync_copy(
            xt_hbm.at[pl.ds(pl.multiple_of(8 * u, 8), 8),
                      pl.ds(b0, 128)], idx_v)
        for r8 in range(8):
            for j in range(8):
                v = idx_v[r8, pl.ds(16 * j, LANES)]
                idxs_v[pl.ds(r8 * 128 + 16 * j, LANES)] = (
                    lax.shift_right_logical(v, 1))

    def fire_gather(r, buf):
        pltpu.async_copy(tp_hbm.at[idxs_v.at[pl.ds(128 * r, 128)]],
                         g_v[buf], sem_g[buf])

    def wait_gather(buf):
        pltpu.make_async_copy(
            tp_hbm.at[pl.ds(0, 128)], g_v[buf], sem_g[buf]).wait()

    def fire_out(s, buf):
        pltpu.async_copy(
            o_v[buf], op_hbm.at[s, :, pl.ds(b0, 128)], sem_o[buf])

    def wait_out(buf):
        pltpu.make_async_copy(
            o_v[buf], op_hbm.at[0, :, pl.ds(b0, 128)], sem_o[buf]).wait()

    def transpose_select(r, gbuf, obuf):
        hvs = [(idx_v[r, pl.ds(16 * j, LANES)] & 1) * 64 for j in range(8)]

        @plsc.parallel_loop(0, LANES, unroll=4)
        def _(d):
            fd = (iota + d) & (LANES - 1)
            for k in range(DIM // LANES):
                fv = fd + 16 * k
                for j in range(8):
                    vals = plsc.load_gather(g_v[gbuf], [rowjs[j], hvs[j] + fv])
                    plsc.store_scatter(o_v[obuf], [fv, rowjs[j]], vals)

    def group(u, carry):
        load_idx_tile(u)
        fire_gather(0, 0)
        fire_gather(1, 1)
        for r in range(8):
            s = 8 * u + r
            if r + 2 < 8:
                fire_gather(r + 2, (r + 2) % 4)
            wait_gather(r % 4)

            @pl.when((u > 0) | (r >= 2))
            def _():
                wait_out(r % 2)

            transpose_select(r, r % 4, r % 2)
            fire_out(s, r % 2)
        return carry

    lax.fori_loop(0, NGRP, group, 0)
    wait_out(0)
    wait_out(1)


def _make_gather():
    return pl.kernel(
        _gather_body,
        out_type=jax.ShapeDtypeStruct((SEQ, DIM, BATCH), jnp.float32),
        mesh=_MESH,
        scratch_types=[
            pltpu.VMEM((8, 128), jnp.int32),
            pltpu.VMEM((1024,), jnp.int32),
            [pltpu.VMEM((128, 128), jnp.float32) for _ in range(4)],
            [pltpu.VMEM((DIM, 128), jnp.float32) for _ in range(2)],
            [pltpu.SemaphoreType.DMA for _ in range(4)],
            [pltpu.SemaphoreType.DMA for _ in range(2)],
        ],
        compiler_params=_TC_TILED,
    )


def kernel(x, weight):
    wt = weight.T
    tail = weight[NFULL * VB:].reshape(TAILV // 2, 128)
    tp = _make_transpose()(wt, tail)
    xt = x.T.astype(jnp.int32)
    op = _make_gather()(xt, tp)
    return op.transpose(2, 0, 1)
